# baseline (speedup 1.0000x reference)
import jax
import jax.numpy as jnp
from jax.experimental import pallas as pl
from jax.experimental.pallas import tpu as pltpu

E = 8
EMB = 128
B = 8
HW = 1024
W_IMG = 32
NP = 4


def _gating_body(x_ref, gt_ref, coef_ref, loss_ref):
    xg = jnp.mean(x_ref[...], axis=2)
    logits = jax.lax.dot_general(
        xg.astype(jnp.bfloat16), gt_ref[...], (((1,), (1,)), ((), ())),
        preferred_element_type=jnp.float32)
    eidx = jax.lax.broadcasted_iota(jnp.int32, (B, E), 1)
    cv_sum = jnp.float32(0.0)
    for g in range(4):
        lg = logits[:, g * E:(g + 1) * E]
        m = jnp.max(lg, axis=1, keepdims=True)
        ex = jnp.exp(lg - m)
        probs = ex / jnp.sum(ex, axis=1, keepdims=True)
        ml1 = jnp.max(lg, axis=1, keepdims=True)
        i1 = jnp.min(jnp.where(lg == ml1, eidx, E), axis=1, keepdims=True)
        l2 = jnp.where(eidx == i1, -jnp.inf, lg)
        ml2 = jnp.max(l2, axis=1, keepdims=True)
        i2 = jnp.min(jnp.where(l2 == ml2, eidx, E), axis=1, keepdims=True)
        m1 = jnp.max(probs, axis=1, keepdims=True)
        m2 = jnp.max(jnp.where(eidx == i1, -1.0, probs), axis=1,
                     keepdims=True)
        e2 = jnp.exp(m2 - m1)
        w1 = 1.0 / (1.0 + e2)
        w2 = e2 / (1.0 + e2)
        coef_ref[g] = (jnp.where(eidx == i1, w1, 0.0)
                       + jnp.where(eidx == i2, w2, 0.0))
        usage = jnp.sum(probs, axis=0)
        mu = jnp.mean(usage)
        var = jnp.sum((usage - mu) ** 2) / (E - 1)
        cv_sum = cv_sum + var / (mu * mu + 1e-10)
    loss_ref[...] = jnp.reshape(cv_sum * 0.25, (1, 1))


def _moe_body(coef_ref, cb_ref, x_ref, wa_ref, wpall_ref, bc_ref, bp_ref,
              o0, o1, o2, o3, xs_ref, v_ref, cw_ref):
    b = pl.program_id(0)

    xf = x_ref[0]
    p = jax.lax.broadcasted_iota(jnp.int32, (EMB, HW), 1)
    xx = p & (W_IMG - 1)
    for ky in range(3):
        for kx in range(3):
            off = (ky - 1) * W_IMG + (kx - 1)
            if off == 0:
                sh = xf
            else:
                sh = pltpu.roll(xf, (-off) % HW, axis=1)
            valid = None
            if kx == 0:
                valid = xx >= 1
            elif kx == 2:
                valid = xx <= W_IMG - 2
            if ky == 0:
                vy = p >= W_IMG
                valid = vy if valid is None else (valid & vy)
            elif ky == 2:
                vy = p < HW - W_IMG
                valid = vy if valid is None else (valid & vy)
            if valid is not None:
                sh = jnp.where(valid, sh, 0.0)
            k = ky * 3 + kx
            xs_ref[k * EMB:(k + 1) * EMB, :] = sh.astype(jnp.bfloat16)

    for j in range(NP):
        c = [coef_ref[g, b, e] for g in range(4)
             for e in (2 * j, 2 * j + 1)]
        tot = sum(jnp.abs(cg) for cg in c)

        @pl.when(tot > 0.0)
        def _compute(j=j):
            u = jnp.dot(wa_ref[j], xs_ref[...],
                        preferred_element_type=jnp.float32)
            u = jnp.maximum(u + bc_ref[j], 0.0)
            for h in range(2):
                uh = u[h * EMB:(h + 1) * EMB]
                sq = jnp.sum(uh * uh, axis=0, keepdims=True)
                scale = sq / (1.0 + sq) * jax.lax.rsqrt(sq + 1e-8)
                r0 = (2 * j + h) * EMB
                v_ref[r0:r0 + EMB, :] = (uh * scale).astype(jnp.bfloat16)

        @pl.when(tot == 0.0)
        def _zero(j=j):
            v_ref[2 * j * EMB:(2 * j + 2) * EMB, :] = jnp.zeros(
                (2 * EMB, HW), dtype=jnp.bfloat16)

    wp = wpall_ref[...]
    for g in range(4):
        cw_ref[g * EMB:(g + 1) * EMB, :] = (
            wp * cb_ref[0, g:g + 1]).astype(jnp.bfloat16)
    res = jnp.dot(cw_ref[...], v_ref[...],
                  preferred_element_type=jnp.float32)
    for g, og in enumerate((o0, o1, o2, o3)):
        bias = sum(coef_ref[g, b, ee] * bp_ref[:, ee:ee + 1]
                   for ee in range(E))
        og[0] = res[g * EMB:(g + 1) * EMB, :] + bias


@jax.jit
def kernel(x, gates, Wc, bc, Wp, bp):
    x3 = x.reshape(B, EMB, HW)
    gt = jnp.transpose(gates, (0, 2, 1)).reshape(4 * E, EMB)
    gt = gt.astype(jnp.bfloat16)

    coef, loss = pl.pallas_call(
        _gating_body,
        out_shape=(
            jax.ShapeDtypeStruct((4, B, E), jnp.float32),
            jax.ShapeDtypeStruct((1, 1), jnp.float32),
        ),
    )(x3, gt)

    cb = jnp.broadcast_to(
        jnp.transpose(coef, (1, 0, 2))[:, :, :, None], (B, 4, E, EMB))
    cb = cb.reshape(B, 4, E * EMB)

    wa = jnp.transpose(Wc.reshape(E, EMB, EMB, 9), (0, 1, 3, 2))
    wa = wa.reshape(NP, 2 * EMB, 9 * EMB).astype(jnp.bfloat16)
    wpall = jnp.transpose(Wp.reshape(E, EMB, EMB), (1, 0, 2))
    wpall = wpall.reshape(EMB, E * EMB)
    bc_t = bc.reshape(NP, 2 * EMB, 1)
    bp_t = jnp.transpose(bp, (1, 0))

    grid = (B,)
    outs = pl.pallas_call(
        _moe_body,
        grid=grid,
        in_specs=[
            pl.BlockSpec(memory_space=pltpu.SMEM),
            pl.BlockSpec((1, 4, E * EMB), lambda b: (b, 0, 0)),
            pl.BlockSpec((1, EMB, HW), lambda b: (b, 0, 0)),
            pl.BlockSpec((NP, 2 * EMB, 9 * EMB), lambda b: (0, 0, 0)),
            pl.BlockSpec((EMB, E * EMB), lambda b: (0, 0)),
            pl.BlockSpec((NP, 2 * EMB, 1), lambda b: (0, 0, 0)),
            pl.BlockSpec((EMB, E), lambda b: (0, 0)),
        ],
        out_specs=tuple(
            pl.BlockSpec((1, EMB, HW), lambda b: (b, 0, 0))
            for _ in range(4)),
        out_shape=tuple(
            jax.ShapeDtypeStruct((B, EMB, HW), jnp.float32)
            for _ in range(4)),
        scratch_shapes=[pltpu.VMEM((9 * EMB, HW), jnp.bfloat16),
                        pltpu.VMEM((E * EMB, HW), jnp.bfloat16),
                        pltpu.VMEM((4 * EMB, HW), jnp.bfloat16)],
        compiler_params=pltpu.CompilerParams(
            dimension_semantics=("arbitrary",)),
    )(coef, cb, x3, wa, wpall, bc_t, bp_t)

    o = tuple(y.reshape(B, EMB, W_IMG, W_IMG) for y in outs)
    return (*o, loss.reshape(()))


# speedup vs baseline: 1.0685x; 1.0685x over previous
import jax
import jax.numpy as jnp
from jax.experimental import pallas as pl
from jax.experimental.pallas import tpu as pltpu

E = 8
EMB = 128
B = 8
HW = 1024
W_IMG = 32
NP = 4


def _gating_body(x_ref, gt_ref, coef_ref, idx_ref, loss_ref):
    xg = jnp.mean(x_ref[...], axis=2)
    logits = jax.lax.dot_general(
        xg.astype(jnp.bfloat16), gt_ref[...], (((1,), (1,)), ((), ())),
        preferred_element_type=jnp.float32)
    eidx = jax.lax.broadcasted_iota(jnp.int32, (B, E), 1)
    cv_sum = jnp.float32(0.0)
    for g in range(4):
        lg = logits[:, g * E:(g + 1) * E]
        m = jnp.max(lg, axis=1, keepdims=True)
        ex = jnp.exp(lg - m)
        probs = ex / jnp.sum(ex, axis=1, keepdims=True)
        ml1 = jnp.max(lg, axis=1, keepdims=True)
        i1 = jnp.min(jnp.where(lg == ml1, eidx, E), axis=1, keepdims=True)
        l2 = jnp.where(eidx == i1, -jnp.inf, lg)
        ml2 = jnp.max(l2, axis=1, keepdims=True)
        i2 = jnp.min(jnp.where(l2 == ml2, eidx, E), axis=1, keepdims=True)
        m1 = jnp.max(probs, axis=1, keepdims=True)
        m2 = jnp.max(jnp.where(eidx == i1, -1.0, probs), axis=1,
                     keepdims=True)
        e2 = jnp.exp(m2 - m1)
        w1 = 1.0 / (1.0 + e2)
        w2 = e2 / (1.0 + e2)
        coef_ref[g] = (jnp.where(eidx == i1, w1, 0.0)
                       + jnp.where(eidx == i2, w2, 0.0))
        idx_ref[g] = jnp.concatenate(
            [i1, i2, jnp.zeros((B, E - 2), jnp.int32)], axis=1)
        usage = jnp.sum(probs, axis=0)
        mu = jnp.mean(usage)
        var = jnp.sum((usage - mu) ** 2) / (E - 1)
        cv_sum = cv_sum + var / (mu * mu + 1e-10)
    loss_ref[...] = jnp.reshape(cv_sum * 0.25, (1, 1))


def _moe_body(coef_ref, idx_ref, x_ref, wa_ref, wp3_ref, bc_ref, bp3_ref,
              o0, o1, o2, o3, xs_ref, v_ref):
    b = pl.program_id(0)

    xf = x_ref[0]
    p = jax.lax.broadcasted_iota(jnp.int32, (EMB, HW), 1)
    xx = p & (W_IMG - 1)
    for ky in range(3):
        for kx in range(3):
            off = (ky - 1) * W_IMG + (kx - 1)
            if off == 0:
                sh = xf
            else:
                sh = pltpu.roll(xf, (-off) % HW, axis=1)
            valid = None
            if kx == 0:
                valid = xx >= 1
            elif kx == 2:
                valid = xx <= W_IMG - 2
            if ky == 0:
                vy = p >= W_IMG
                valid = vy if valid is None else (valid & vy)
            elif ky == 2:
                vy = p < HW - W_IMG
                valid = vy if valid is None else (valid & vy)
            if valid is not None:
                sh = jnp.where(valid, sh, 0.0)
            k = ky * 3 + kx
            xs_ref[k * EMB:(k + 1) * EMB, :] = sh.astype(jnp.bfloat16)

    for j in range(NP):
        c = [coef_ref[g, b, e] for g in range(4)
             for e in (2 * j, 2 * j + 1)]
        tot = sum(jnp.abs(cg) for cg in c)

        @pl.when(tot > 0.0)
        def _compute(j=j):
            u = jnp.dot(wa_ref[j], xs_ref[...],
                        preferred_element_type=jnp.float32)
            u = jnp.maximum(u + bc_ref[j], 0.0)
            for h in range(2):
                uh = u[h * EMB:(h + 1) * EMB]
                sq = jnp.sum(uh * uh, axis=0, keepdims=True)
                scale = sq / (1.0 + sq) * jax.lax.rsqrt(sq + 1e-8)
                r0 = (2 * j + h) * EMB
                v_ref[r0:r0 + EMB, :] = (uh * scale).astype(jnp.bfloat16)

        @pl.when(tot == 0.0)
        def _zero(j=j):
            v_ref[2 * j * EMB:(2 * j + 2) * EMB, :] = jnp.zeros(
                (2 * EMB, HW), dtype=jnp.bfloat16)

    for g, og in enumerate((o0, o1, o2, o3)):
        acc = None
        for t in range(2):
            e = idx_ref[g, b, t]
            wgt = coef_ref[g, b, e]
            vp = v_ref[pl.ds(e * EMB, EMB), :]
            part = jnp.dot(wp3_ref[e], vp,
                           preferred_element_type=jnp.float32)
            term = wgt * part + wgt * bp3_ref[e]
            acc = term if acc is None else acc + term
        og[0] = acc


@jax.jit
def kernel(x, gates, Wc, bc, Wp, bp):
    x3 = x.reshape(B, EMB, HW)
    gt = jnp.transpose(gates, (0, 2, 1)).reshape(4 * E, EMB)
    gt = gt.astype(jnp.bfloat16)

    coef, idx, loss = pl.pallas_call(
        _gating_body,
        out_shape=(
            jax.ShapeDtypeStruct((4, B, E), jnp.float32),
            jax.ShapeDtypeStruct((4, B, E), jnp.int32),
            jax.ShapeDtypeStruct((1, 1), jnp.float32),
        ),
    )(x3, gt)

    wa = jnp.transpose(Wc.reshape(E, EMB, EMB, 9), (0, 1, 3, 2))
    wa = wa.reshape(NP, 2 * EMB, 9 * EMB).astype(jnp.bfloat16)
    wp3 = Wp.reshape(E, EMB, EMB).astype(jnp.bfloat16)
    bc_t = bc.reshape(NP, 2 * EMB, 1)
    bp3 = bp.reshape(E, EMB, 1)

    grid = (B,)
    outs = pl.pallas_call(
        _moe_body,
        grid=grid,
        in_specs=[
            pl.BlockSpec(memory_space=pltpu.SMEM),
            pl.BlockSpec(memory_space=pltpu.SMEM),
            pl.BlockSpec((1, EMB, HW), lambda b: (b, 0, 0)),
            pl.BlockSpec((NP, 2 * EMB, 9 * EMB), lambda b: (0, 0, 0)),
            pl.BlockSpec((E, EMB, EMB), lambda b: (0, 0, 0)),
            pl.BlockSpec((NP, 2 * EMB, 1), lambda b: (0, 0, 0)),
            pl.BlockSpec((E, EMB, 1), lambda b: (0, 0, 0)),
        ],
        out_specs=tuple(
            pl.BlockSpec((1, EMB, HW), lambda b: (b, 0, 0))
            for _ in range(4)),
        out_shape=tuple(
            jax.ShapeDtypeStruct((B, EMB, HW), jnp.float32)
            for _ in range(4)),
        scratch_shapes=[pltpu.VMEM((9 * EMB, HW), jnp.bfloat16),
                        pltpu.VMEM((E * EMB, HW), jnp.bfloat16)],
        compiler_params=pltpu.CompilerParams(
            dimension_semantics=("arbitrary",)),
    )(coef, idx, x3, wa, wp3, bc_t, bp3)

    o = tuple(y.reshape(B, EMB, W_IMG, W_IMG) for y in outs)
    return (*o, loss.reshape(()))
